# baseline (speedup 1.0000x reference)
import functools

import jax
import jax.numpy as jnp
from jax import lax
from jax.experimental import pallas as pl
from jax.experimental.pallas import tpu as pltpu
from jax.experimental.pallas import tpu_sc as plsc

_TILE = 16384
_CHUNK = 256
_LOG2E = 1.4426950408889634
_LN2 = 0.6931471805599453


def _make_sc_gather(b, d):
    info = plsc.get_sparse_core_info()
    nc, ns = info.num_cores, info.num_subcores
    nw = nc * ns
    bpw = b // nw
    mesh = plsc.VectorSubcoreMesh(core_axis_name="c", subcore_axis_name="s")

    @functools.partial(
        pl.kernel,
        mesh=mesh,
        out_type=jax.ShapeDtypeStruct((b, d), jnp.float32),
        scratch_types=[
            pltpu.VMEM((bpw,), jnp.int32),
            pltpu.VMEM((bpw, d), jnp.float32),
            pltpu.SemaphoreType.DMA,
        ],
    )
    def gather_kernel(table_hbm, idx_hbm, out_hbm, idx_v, rows_v, sem):
        wid = lax.axis_index("s") * nc + lax.axis_index("c")
        base = wid * bpw
        pltpu.sync_copy(idx_hbm.at[pl.ds(base, bpw)], idx_v)
        pltpu.async_copy(table_hbm.at[idx_v], rows_v, sem).wait()
        pltpu.sync_copy(rows_v, out_hbm.at[pl.ds(base, bpw)])

    return gather_kernel


def _fold_chunk(scores, m_ref, s_ref):
    bm = jnp.max(scores, axis=0, keepdims=True)
    ex = jnp.exp2(scores - bm)
    ones = jnp.ones((1, scores.shape[0]), jnp.bfloat16)
    t = lax.dot_general(
        ones, ex, (((1,), (0,)), ((), ())), preferred_element_type=jnp.float32
    )
    bm32 = bm.astype(jnp.float32)
    m_old = m_ref[...]
    m_new = jnp.maximum(m_old, bm32)
    s_ref[...] = s_ref[...] * jnp.exp2(m_old - m_new) + t * jnp.exp2(
        bm32 - m_new
    )
    m_ref[...] = m_new


def _logz_body(qt_ref, e_ref, m_out, s_out, m_ref, s_ref):
    i = pl.program_id(0)
    nsteps = pl.num_programs(0)

    @pl.when(i == 0)
    def _():
        m_ref[...] = jnp.full(m_ref.shape, -jnp.inf, m_ref.dtype)
        s_ref[...] = jnp.zeros(s_ref.shape, s_ref.dtype)

    for c in range(_TILE // _CHUNK):
        scores = lax.dot_general(
            e_ref[pl.ds(c * _CHUNK, _CHUNK), :].astype(jnp.bfloat16),
            qt_ref[...],
            (((1,), (0,)), ((), ())),
            preferred_element_type=jnp.float32,
        ).astype(jnp.bfloat16)
        _fold_chunk(scores, m_ref, s_ref)

    @pl.when(i == nsteps - 1)
    def _():
        m_out[...] = m_ref[...]
        s_out[...] = s_ref[...]


def _combine_body(qt_ref, et_ref, q_ref, g_ref, m_in, s_in, out_ref, m_ref, s_ref):
    m_ref[...] = m_in[...]
    s_ref[...] = s_in[...]
    scores = lax.dot_general(
        et_ref[...].astype(jnp.bfloat16),
        qt_ref[...],
        (((1,), (0,)), ((), ())),
        preferred_element_type=jnp.float32,
    ).astype(jnp.bfloat16)
    _fold_chunk(scores, m_ref, s_ref)
    logz = m_ref[...] * _LN2 + jnp.log(s_ref[...])
    b = q_ref.shape[0]
    loss = (jnp.sum(logz) - jnp.sum(q_ref[...] * g_ref[...])) / b
    out_ref[...] = jnp.full((1, 1), loss, out_ref.dtype)


def kernel(query_embs, ent_embs, triplets):
    b, h = query_embs.shape
    n = ent_embs.shape[0]
    labels = triplets[:, 2].astype(jnp.int32)

    g = _make_sc_gather(b, h)(ent_embs, labels)

    ntiles = n // _TILE
    tail = n - ntiles * _TILE
    qt = (query_embs.T * _LOG2E).astype(jnp.bfloat16)

    m, s = pl.pallas_call(
        _logz_body,
        grid=(ntiles,),
        in_specs=[
            pl.BlockSpec((h, b), lambda i: (0, 0)),
            pl.BlockSpec((_TILE, h), lambda i: (i, 0)),
        ],
        out_specs=[
            pl.BlockSpec((1, b), lambda i: (0, 0)),
            pl.BlockSpec((1, b), lambda i: (0, 0)),
        ],
        out_shape=[
            jax.ShapeDtypeStruct((1, b), jnp.float32),
            jax.ShapeDtypeStruct((1, b), jnp.float32),
        ],
        scratch_shapes=[
            pltpu.VMEM((1, b), jnp.float32),
            pltpu.VMEM((1, b), jnp.float32),
        ],
    )(qt, ent_embs)

    loss = pl.pallas_call(
        _combine_body,
        out_shape=jax.ShapeDtypeStruct((1, 1), jnp.float32),
        scratch_shapes=[
            pltpu.VMEM((1, b), jnp.float32),
            pltpu.VMEM((1, b), jnp.float32),
        ],
    )(qt, ent_embs[ntiles * _TILE :], query_embs, g, m, s)
    return loss[0, 0]


# speedup vs baseline: 1.7067x; 1.7067x over previous
import functools

import jax
import jax.numpy as jnp
from jax import lax
from jax.experimental import pallas as pl
from jax.experimental.pallas import tpu as pltpu
from jax.experimental.pallas import tpu_sc as plsc

_TILE = 16384
_CHUNK = 256
_LOG2E = 1.4426950408889634
_LN2 = 0.6931471805599453


def _make_sc_gather(b, d):
    info = plsc.get_sparse_core_info()
    nc, ns = info.num_cores, info.num_subcores
    nw = nc * ns
    bpw = b // nw
    mesh = plsc.VectorSubcoreMesh(core_axis_name="c", subcore_axis_name="s")

    @functools.partial(
        pl.kernel,
        mesh=mesh,
        out_type=jax.ShapeDtypeStruct((b, d), jnp.float32),
        scratch_types=[
            pltpu.VMEM((bpw,), jnp.int32),
            pltpu.VMEM((bpw, d), jnp.float32),
            pltpu.SemaphoreType.DMA,
        ],
    )
    def gather_kernel(table_hbm, idx_hbm, out_hbm, idx_v, rows_v, sem):
        wid = lax.axis_index("s") * nc + lax.axis_index("c")
        base = wid * bpw
        pltpu.sync_copy(idx_hbm.at[pl.ds(base, bpw)], idx_v)
        pltpu.async_copy(table_hbm.at[idx_v], rows_v, sem).wait()
        pltpu.sync_copy(rows_v, out_hbm.at[pl.ds(base, bpw)])

    return gather_kernel


def _fold_chunk(scores, m_ref, s_ref):
    bm = jnp.max(scores, axis=0, keepdims=True)
    t = jnp.sum(
        jnp.exp2(scores - bm),
        axis=0,
        keepdims=True,
        dtype=jnp.bfloat16,
    ).astype(jnp.float32)
    bm32 = bm.astype(jnp.float32)
    m_old = m_ref[...]
    m_new = jnp.maximum(m_old, bm32)
    s_ref[...] = s_ref[...] * jnp.exp2(m_old - m_new) + t * jnp.exp2(
        bm32 - m_new
    )
    m_ref[...] = m_new


def _logz_body(qt_ref, e_ref, m_out, s_out, m_ref, s_ref):
    i = pl.program_id(0)
    nsteps = pl.num_programs(0)

    @pl.when(i == 0)
    def _():
        m_ref[...] = jnp.full(m_ref.shape, -jnp.inf, m_ref.dtype)
        s_ref[...] = jnp.zeros(s_ref.shape, s_ref.dtype)

    for c in range(_TILE // _CHUNK):
        scores = lax.dot_general(
            e_ref[pl.ds(c * _CHUNK, _CHUNK), :].astype(jnp.bfloat16),
            qt_ref[...],
            (((1,), (0,)), ((), ())),
            preferred_element_type=jnp.float32,
        ).astype(jnp.bfloat16)
        _fold_chunk(scores, m_ref, s_ref)

    @pl.when(i == nsteps - 1)
    def _():
        m_out[...] = m_ref[...]
        s_out[...] = s_ref[...]


def _combine_body(qt_ref, et_ref, q_ref, g_ref, m_in, s_in, out_ref, m_ref, s_ref):
    m_ref[...] = m_in[...]
    s_ref[...] = s_in[...]
    scores = lax.dot_general(
        et_ref[...].astype(jnp.bfloat16),
        qt_ref[...],
        (((1,), (0,)), ((), ())),
        preferred_element_type=jnp.float32,
    ).astype(jnp.bfloat16)
    _fold_chunk(scores, m_ref, s_ref)
    logz = m_ref[...] * _LN2 + jnp.log(s_ref[...])
    b = q_ref.shape[0]
    loss = (jnp.sum(logz) - jnp.sum(q_ref[...] * g_ref[...])) / b
    out_ref[...] = jnp.full((1, 1), loss, out_ref.dtype)


def kernel(query_embs, ent_embs, triplets):
    b, h = query_embs.shape
    n = ent_embs.shape[0]
    labels = triplets[:, 2].astype(jnp.int32)

    g = _make_sc_gather(b, h)(ent_embs, labels)

    ntiles = n // _TILE
    tail = n - ntiles * _TILE
    qt = (query_embs.T * _LOG2E).astype(jnp.bfloat16)

    m, s = pl.pallas_call(
        _logz_body,
        grid=(ntiles,),
        in_specs=[
            pl.BlockSpec((h, b), lambda i: (0, 0)),
            pl.BlockSpec((_TILE, h), lambda i: (i, 0)),
        ],
        out_specs=[
            pl.BlockSpec((1, b), lambda i: (0, 0)),
            pl.BlockSpec((1, b), lambda i: (0, 0)),
        ],
        out_shape=[
            jax.ShapeDtypeStruct((1, b), jnp.float32),
            jax.ShapeDtypeStruct((1, b), jnp.float32),
        ],
        scratch_shapes=[
            pltpu.VMEM((1, b), jnp.float32),
            pltpu.VMEM((1, b), jnp.float32),
        ],
    )(qt, ent_embs)

    loss = pl.pallas_call(
        _combine_body,
        out_shape=jax.ShapeDtypeStruct((1, 1), jnp.float32),
        scratch_shapes=[
            pltpu.VMEM((1, b), jnp.float32),
            pltpu.VMEM((1, b), jnp.float32),
        ],
    )(qt, ent_embs[ntiles * _TILE :], query_embs, g, m, s)
    return loss[0, 0]
